# baseline (speedup 1.0000x reference)
import functools
import jax
import jax.numpy as jnp
from jax import lax
from jax.experimental import pallas as pl
from jax.experimental.pallas import tpu as pltpu
from jax.experimental.pallas import tpu_sc as plsc

N = 10000
E = 160000
G = 128
EPS = 1e-5
F32 = jnp.float32
LW = 128
BR = 2000
NB = N // BR
K_EDGE = 80
NS = 16


def _sc_agg_body(C, n_rows, n_copy_tiles, rpc, e_pad, per_tile,
                 p_hbm, srcp, dstp, zeros_hbm, out, src_v, dst_v, rows_v, sem, acc):
    core = lax.axis_index("c")
    tile = lax.axis_index("s")
    row0 = tile * rpc
    nj = per_tile // K_EDGE

    def process(cc):
        @pl.when(tile < n_copy_tiles)
        def _():
            pltpu.sync_copy(zeros_hbm, acc.at[pl.ds(row0, rpc)])
        plsc.subcore_barrier()
        e0 = tile * per_tile

        def step(j, carry):
            base = e0 + j * K_EDGE
            pltpu.sync_copy(srcp.at[pl.ds(base, K_EDGE)], src_v)
            pltpu.sync_copy(dstp.at[pl.ds(base, K_EDGE)], dst_v)
            pltpu.async_copy(p_hbm.at[cc].at[src_v], rows_v, sem).wait()
            pltpu.sync_copy(rows_v, acc.at[dst_v], add=True)
            return carry

        lax.fori_loop(0, nj, step, 0)
        plsc.subcore_barrier()

        @pl.when(tile < n_copy_tiles)
        def _():
            pltpu.sync_copy(acc.at[pl.ds(row0, rpc)],
                            out.at[cc].at[pl.ds(row0, rpc)])
        plsc.subcore_barrier()

    for idx in range((C + 1) // 2):
        for co in (0, 1):
            cc = idx * 2 + co
            if cc >= C:
                continue

            @pl.when(core == co)
            def _(cc=cc):
                process(cc)


def _make_sc_agg(C, n_rows, n_copy_tiles, rpc, e_pad):
    per_tile = e_pad // NS
    mesh = plsc.VectorSubcoreMesh(core_axis_name="c", subcore_axis_name="s")
    body = functools.partial(_sc_agg_body, C, n_rows, n_copy_tiles, rpc, e_pad, per_tile)
    return pl.kernel(
        body,
        out_type=jax.ShapeDtypeStruct((C, n_rows, LW), F32),
        mesh=mesh,
        scratch_types=[
            pltpu.VMEM((K_EDGE,), jnp.int32),
            pltpu.VMEM((K_EDGE,), jnp.int32),
            pltpu.VMEM((K_EDGE, LW), F32),
            pltpu.SemaphoreType.DMA,
            pltpu.VMEM_SHARED((n_rows, LW), F32),
        ],
    )


def _mm_scale_body(first, din, x_ref, m_ref, v_ref, g_ref, be_ref, w_ref,
                   dinv_ref, nself_ref, p_ref, self_ref):
    xb = x_ref[...]
    if not first:
        t = g_ref[...] * (xb - m_ref[...])
        xb = t * lax.rsqrt(v_ref[...] + EPS) + be_ref[...]
    hw = jnp.dot(xb, w_ref[...], preferred_element_type=F32)
    p_ref[0] = dinv_ref[...] * hw
    self_ref[0] = hw * nself_ref[...]


def _mm_scale(Y, m, v, g, be, W, dinv, nself, first):
    din, dout = W.shape
    C = dout // LW
    grid = (NB, C)
    in_specs = [pl.BlockSpec((BR, din), lambda i, c: (i, 0)),
                pl.BlockSpec((1, din), lambda i, c: (0, 0)),
                pl.BlockSpec((1, din), lambda i, c: (0, 0)),
                pl.BlockSpec((1, din), lambda i, c: (0, 0)),
                pl.BlockSpec((1, din), lambda i, c: (0, 0)),
                pl.BlockSpec((din, LW), lambda i, c: (0, c)),
                pl.BlockSpec((BR, 1), lambda i, c: (i, 0)),
                pl.BlockSpec((BR, 1), lambda i, c: (i, 0))]
    out_specs = [pl.BlockSpec((1, BR, LW), lambda i, c: (c, i, 0)),
                 pl.BlockSpec((1, BR, LW), lambda i, c: (c, i, 0))]
    out_shape = [jax.ShapeDtypeStruct((C, N, LW), F32),
                 jax.ShapeDtypeStruct((C, N, LW), F32)]
    return pl.pallas_call(
        functools.partial(_mm_scale_body, first, din),
        grid=grid, in_specs=in_specs, out_specs=out_specs, out_shape=out_shape,
    )(Y, m, v, g, be, W, dinv, nself)


def _combine_body(q_ref, s_ref, dinv_ref, b_ref, y_ref, s1_ref):
    i = pl.program_id(1)
    t = dinv_ref[...] * q_ref[0]
    t = t + s_ref[0]
    y = jnp.maximum(t + b_ref[...], 0.0)
    y_ref[...] = y
    ps = jnp.sum(y, axis=0, keepdims=True)

    @pl.when(i == 0)
    def _():
        s1_ref[...] = ps

    @pl.when(i > 0)
    def _():
        s1_ref[...] = s1_ref[...] + ps


def _combine(q, selft, dinv, b):
    C = q.shape[0]
    dout = C * LW
    grid = (C, NB)
    in_specs = [pl.BlockSpec((1, BR, LW), lambda c, i: (c, i, 0)),
                pl.BlockSpec((1, BR, LW), lambda c, i: (c, i, 0)),
                pl.BlockSpec((BR, 1), lambda c, i: (i, 0)),
                pl.BlockSpec((1, LW), lambda c, i: (0, c))]
    out_specs = [pl.BlockSpec((BR, LW), lambda c, i: (i, c)),
                 pl.BlockSpec((1, LW), lambda c, i: (0, c))]
    out_shape = [jax.ShapeDtypeStruct((N, dout), F32),
                 jax.ShapeDtypeStruct((1, dout), F32)]
    return pl.pallas_call(
        _combine_body, grid=grid, in_specs=in_specs, out_specs=out_specs,
        out_shape=out_shape,
    )(q, selft, dinv, b.reshape(1, dout))


def _var_body(y_ref, m_ref, s2_ref):
    i = pl.program_id(1)
    d = y_ref[...] - m_ref[...]
    ps = jnp.sum(jnp.abs(d) ** 2, axis=0, keepdims=True)

    @pl.when(i == 0)
    def _():
        s2_ref[...] = ps

    @pl.when(i > 0)
    def _():
        s2_ref[...] = s2_ref[...] + ps


def _var_sums(Y, m):
    dout = Y.shape[1]
    C = dout // LW
    grid = (C, NB)
    return pl.pallas_call(
        _var_body, grid=grid,
        in_specs=[pl.BlockSpec((BR, LW), lambda c, i: (i, c)),
                  pl.BlockSpec((1, LW), lambda c, i: (0, c))],
        out_specs=pl.BlockSpec((1, LW), lambda c, i: (0, c)),
        out_shape=jax.ShapeDtypeStruct((1, dout), F32),
    )(Y, m)


def _gate_body(y_ref, m_ref, v_ref, g_ref, be_ref, gw_ref, gb_ref, b_ref,
               h_ref, gate_ref, mg_ref):
    i = pl.program_id(0)
    t = g_ref[...] * (y_ref[...] - m_ref[...])
    hb = t * lax.rsqrt(v_ref[...] + EPS) + be_ref[...]
    h_ref[...] = hb
    gate = jnp.dot(hb, gw_ref[...], preferred_element_type=F32) + gb_ref[...]
    gate_ref[...] = gate
    iota = lax.broadcasted_iota(jnp.int32, (BR, G), 1)
    mask = b_ref[...] == iota
    gv = jnp.where(mask, gate, -jnp.inf)
    bm = jnp.max(gv, axis=0, keepdims=True)

    @pl.when(i == 0)
    def _():
        mg_ref[...] = bm

    @pl.when(i > 0)
    def _():
        mg_ref[...] = jnp.maximum(mg_ref[...], bm)


def _gate(Y5, m, v, g, be, gW, gb, batch2):
    grid = (NB,)
    in_specs = [pl.BlockSpec((BR, 1024), lambda i: (i, 0)),
                pl.BlockSpec((1, 1024), lambda i: (0, 0)),
                pl.BlockSpec((1, 1024), lambda i: (0, 0)),
                pl.BlockSpec((1, 1024), lambda i: (0, 0)),
                pl.BlockSpec((1, 1024), lambda i: (0, 0)),
                pl.BlockSpec((1024, 1), lambda i: (0, 0)),
                pl.BlockSpec((1, 1), lambda i: (0, 0)),
                pl.BlockSpec((BR, 1), lambda i: (i, 0))]
    out_specs = [pl.BlockSpec((BR, 1024), lambda i: (i, 0)),
                 pl.BlockSpec((BR, 1), lambda i: (i, 0)),
                 pl.BlockSpec((1, G), lambda i: (0, 0))]
    out_shape = [jax.ShapeDtypeStruct((N, 1024), F32),
                 jax.ShapeDtypeStruct((N, 1), F32),
                 jax.ShapeDtypeStruct((1, G), F32)]
    return pl.pallas_call(
        _gate_body, grid=grid, in_specs=in_specs, out_specs=out_specs,
        out_shape=out_shape,
    )(Y5, m, v, g, be, gW, gb.reshape(1, 1), batch2)


def _exp_body(gate_ref, b_ref, mg_ref, e_ref):
    iota = lax.broadcasted_iota(jnp.int32, (BR, G), 1)
    mask = b_ref[...] == iota
    msel = jnp.max(jnp.where(mask, mg_ref[...], -jnp.inf), axis=1, keepdims=True)
    e_ref[...] = jnp.exp(gate_ref[...] - msel)


def _exp(gate, batch2, mg):
    return pl.pallas_call(
        _exp_body, grid=(NB,),
        in_specs=[pl.BlockSpec((BR, 1), lambda i: (i, 0)),
                  pl.BlockSpec((BR, 1), lambda i: (i, 0)),
                  pl.BlockSpec((1, G), lambda i: (0, 0))],
        out_specs=pl.BlockSpec((BR, 1), lambda i: (i, 0)),
        out_shape=jax.ShapeDtypeStruct((N, 1), F32),
    )(gate, batch2, mg)


def _wvals_body(e_ref, b_ref, d_ref, h_ref, o_ref):
    iota = lax.broadcasted_iota(jnp.int32, (BR, G), 1)
    mask = b_ref[...] == iota
    dsel = jnp.max(jnp.where(mask, d_ref[...], -jnp.inf), axis=1, keepdims=True)
    a = e_ref[...] / dsel
    o_ref[0] = a * h_ref[...]


def _wvals(e, batch2, d, h5):
    C = 1024 // LW
    grid = (NB, C)
    return pl.pallas_call(
        _wvals_body, grid=grid,
        in_specs=[pl.BlockSpec((BR, 1), lambda i, c: (i, 0)),
                  pl.BlockSpec((BR, 1), lambda i, c: (i, 0)),
                  pl.BlockSpec((1, G), lambda i, c: (0, 0)),
                  pl.BlockSpec((BR, LW), lambda i, c: (i, c))],
        out_specs=pl.BlockSpec((1, BR, LW), lambda i, c: (c, i, 0)),
        out_shape=jax.ShapeDtypeStruct((C, N, LW), F32),
    )(e, batch2, d, h5)


def _mlp_body(p_ref, w6_ref, b6_ref, w7_ref, b7_ref, w8_ref, b8_ref, o_ref):
    r = jnp.maximum(jnp.dot(p_ref[...], w6_ref[...],
                            preferred_element_type=F32) + b6_ref[...], 0.0)
    r = jnp.maximum(jnp.dot(r, w7_ref[...],
                            preferred_element_type=F32) + b7_ref[...], 0.0)
    o_ref[...] = jnp.dot(r, w8_ref[...], preferred_element_type=F32) + b8_ref[...]


def _mlp(pooled, W6, b6, W7, b7, W8, b8):
    full = lambda shape: pl.BlockSpec(shape, lambda: tuple(0 for _ in shape))
    return pl.pallas_call(
        _mlp_body, grid=(),
        in_specs=[full((G, 1024)), full((1024, 128)), full((1, 128)),
                  full((128, 16)), full((1, 16)), full((16, 1)), full((1, 1))],
        out_specs=full((G, 1)),
        out_shape=jax.ShapeDtypeStruct((G, 1), F32),
    )(pooled, W6, b6.reshape(1, 128), W7, b7.reshape(1, 16), W8,
      b8.reshape(1, 1))


NA_PAD = 10240
NA_ROWS = 144
NCT_MAIN, RPC_MAIN = 10, 1000
NCT_ATT, RPC_ATT = 9, 16


def kernel(x, edge_index, batch, W1, b1, g1, be1, W2, b2, g2, be2, W3, b3, g3,
           be3, W4, b4, g4, be4, W5, b5, g5, be5, gW, gb, W6, b6, W7, b7, W8, b8):
    src, dst = edge_index[0], edge_index[1]
    order = jnp.argsort(dst, stable=True)
    dstp = dst[order]
    srcp = src[order]
    seg_start = jnp.searchsorted(dstp, jnp.arange(N + 1, dtype=jnp.int32))
    deg = (seg_start[1:] - seg_start[:-1] + 1).astype(F32)
    dinv = 1.0 / jnp.sqrt(deg)
    dinv2 = dinv[:, None]
    nself = (dinv * dinv)[:, None]

    zeros_main = jnp.zeros((RPC_MAIN, LW), F32)
    zeros_att = jnp.zeros((RPC_ATT, LW), F32)

    params = [(W1, b1, g1, be1), (W2, b2, g2, be2), (W3, b3, g3, be3),
              (W4, b4, g4, be4), (W5, b5, g5, be5)]
    Y = x
    m = v = gg = bb = None
    for li, (W, b, g, be) in enumerate(params):
        first = li == 0
        dzero = jnp.zeros((1, W.shape[0]), F32)
        p, selft = _mm_scale(Y,
                             dzero if first else m,
                             dzero if first else v,
                             dzero if first else gg.reshape(1, -1),
                             dzero if first else bb.reshape(1, -1),
                             W, dinv2, nself, first)
        C = W.shape[1] // LW
        agg = _make_sc_agg(C, N, NCT_MAIN, RPC_MAIN, E)
        q = agg(p, srcp, dstp, zeros_main)
        Y, S1 = _combine(q, selft, dinv2, b)
        m = S1 / float(N)
        S2 = _var_sums(Y, m)
        v = S2 / float(N)
        gg, bb = g, be

    batch2 = batch[:, None]
    h5, gate, mg = _gate(Y, m, v, gg.reshape(1, -1), bb.reshape(1, -1),
                         gW, gb, batch2)
    e = _exp(gate, batch2, mg)

    batch_pad = jnp.concatenate(
        [batch, jnp.full((NA_PAD - N,), NA_ROWS - 1, batch.dtype)])
    iota_pad = jnp.arange(NA_PAD, dtype=jnp.int32)

    e_wide = jnp.pad(jnp.broadcast_to(e, (N, LW)), ((0, NA_PAD - N), (0, 0)))
    agg_d = _make_sc_agg(1, NA_ROWS, NCT_ATT, RPC_ATT, NA_PAD)
    dsum = agg_d(e_wide[None], iota_pad, batch_pad, zeros_att)
    d_row = dsum[0, :G, :1].reshape(1, G)

    wv = _wvals(e, batch2, d_row, h5)
    wv_pad = jnp.pad(wv, ((0, 0), (0, NA_PAD - N), (0, 0)))
    agg_s = _make_sc_agg(1024 // LW, NA_ROWS, NCT_ATT, RPC_ATT, NA_PAD)
    s = agg_s(wv_pad, iota_pad, batch_pad, zeros_att)
    pooled = jnp.transpose(s[:, :G, :], (1, 0, 2)).reshape(G, 1024)

    return _mlp(pooled, W6, b6, W7, b7, W8, b8)


# speedup vs baseline: 1.2927x; 1.2927x over previous
import functools
import jax
import jax.numpy as jnp
from jax import lax
from jax.experimental import pallas as pl
from jax.experimental.pallas import tpu as pltpu
from jax.experimental.pallas import tpu_sc as plsc

N = 10000
E = 160000
G = 128
EPS = 1e-5
F32 = jnp.float32
LW = 128
BR = 2000
NB = N // BR
K_EDGE = 80
NS = 16


def _sc_agg_body(C, n_rows, n_copy_tiles, rpc, e_pad, per_tile,
                 p_hbm, srcp3, dstp3, zeros_hbm, out,
                 src_t, dst0, dst1, rows0, rows1, sem0, sem1, semd0, semd1, acc):
    core = lax.axis_index("c")
    tile = lax.axis_index("s")
    row0 = tile * rpc
    nj = per_tile // K_EDGE
    rows = (rows0, rows1)
    sems = (sem0, sem1)
    dsts = (dst0, dst1)
    semd = (semd0, semd1)

    pltpu.sync_copy(srcp3.at[tile], src_t)

    def process(cc):
        @pl.when(tile < n_copy_tiles)
        def _():
            pltpu.sync_copy(zeros_hbm, acc.at[pl.ds(row0, rpc)])
        plsc.subcore_barrier()

        def issue(j, b):
            pltpu.async_copy(p_hbm.at[cc].at[src_t.at[j]], rows[b], sems[b])
            pltpu.async_copy(dstp3.at[tile].at[j], dsts[b], semd[b])

        def wait(b):
            pltpu.make_async_copy(p_hbm.at[cc].at[src_t.at[0]],
                                  rows[b], sems[b]).wait()
            pltpu.make_async_copy(dstp3.at[tile].at[0], dsts[b], semd[b]).wait()

        def step(j, b):
            @pl.when(j + 1 < nj)
            def _():
                issue(j + 1, 1 - b)
            wait(b)
            pltpu.sync_copy(rows[b], acc.at[dsts[b]], add=True)

        issue(0, 0)

        def dbl(i, carry):
            step(i * 2, 0)
            step(i * 2 + 1, 1)
            return carry

        lax.fori_loop(0, nj // 2, dbl, 0)
        if nj % 2:
            step(nj - 1, 0)
        plsc.subcore_barrier()

        @pl.when(tile < n_copy_tiles)
        def _():
            pltpu.sync_copy(acc.at[pl.ds(row0, rpc)],
                            out.at[cc].at[pl.ds(row0, rpc)])
        plsc.subcore_barrier()

    for idx in range((C + 1) // 2):
        for co in (0, 1):
            cc = idx * 2 + co
            if cc >= C:
                continue

            @pl.when(core == co)
            def _(cc=cc):
                process(cc)


def _make_sc_agg(C, n_rows, n_copy_tiles, rpc, e_pad):
    per_tile = e_pad // NS
    nj = per_tile // K_EDGE
    mesh = plsc.VectorSubcoreMesh(core_axis_name="c", subcore_axis_name="s")
    body = functools.partial(_sc_agg_body, C, n_rows, n_copy_tiles, rpc, e_pad, per_tile)
    return pl.kernel(
        body,
        out_type=jax.ShapeDtypeStruct((C, n_rows, LW), F32),
        mesh=mesh,
        scratch_types=[
            pltpu.VMEM((nj, K_EDGE), jnp.int32),
            pltpu.VMEM((K_EDGE,), jnp.int32),
            pltpu.VMEM((K_EDGE,), jnp.int32),
            pltpu.VMEM((K_EDGE, LW), F32),
            pltpu.VMEM((K_EDGE, LW), F32),
            pltpu.SemaphoreType.DMA,
            pltpu.SemaphoreType.DMA,
            pltpu.SemaphoreType.DMA,
            pltpu.SemaphoreType.DMA,
            pltpu.VMEM_SHARED((n_rows, LW), F32),
        ],
    )


def _mm_scale_body(first, din, x_ref, m_ref, v_ref, g_ref, be_ref, w_ref,
                   dinv_ref, nself_ref, p_ref, self_ref):
    xb = x_ref[...]
    if not first:
        t = g_ref[...] * (xb - m_ref[...])
        xb = t * lax.rsqrt(v_ref[...] + EPS) + be_ref[...]
    hw = jnp.dot(xb, w_ref[...], preferred_element_type=F32)
    p_ref[0] = dinv_ref[...] * hw
    self_ref[0] = hw * nself_ref[...]


def _mm_scale(Y, m, v, g, be, W, dinv, nself, first):
    din, dout = W.shape
    C = dout // LW
    grid = (NB, C)
    in_specs = [pl.BlockSpec((BR, din), lambda i, c: (i, 0)),
                pl.BlockSpec((1, din), lambda i, c: (0, 0)),
                pl.BlockSpec((1, din), lambda i, c: (0, 0)),
                pl.BlockSpec((1, din), lambda i, c: (0, 0)),
                pl.BlockSpec((1, din), lambda i, c: (0, 0)),
                pl.BlockSpec((din, LW), lambda i, c: (0, c)),
                pl.BlockSpec((BR, 1), lambda i, c: (i, 0)),
                pl.BlockSpec((BR, 1), lambda i, c: (i, 0))]
    out_specs = [pl.BlockSpec((1, BR, LW), lambda i, c: (c, i, 0)),
                 pl.BlockSpec((1, BR, LW), lambda i, c: (c, i, 0))]
    out_shape = [jax.ShapeDtypeStruct((C, N, LW), F32),
                 jax.ShapeDtypeStruct((C, N, LW), F32)]
    return pl.pallas_call(
        functools.partial(_mm_scale_body, first, din),
        grid=grid, in_specs=in_specs, out_specs=out_specs, out_shape=out_shape,
    )(Y, m, v, g, be, W, dinv, nself)


def _combine_body(q_ref, s_ref, dinv_ref, b_ref, y_ref, s1_ref):
    i = pl.program_id(1)
    t = dinv_ref[...] * q_ref[0]
    t = t + s_ref[0]
    y = jnp.maximum(t + b_ref[...], 0.0)
    y_ref[...] = y
    ps = jnp.sum(y, axis=0, keepdims=True)

    @pl.when(i == 0)
    def _():
        s1_ref[...] = ps

    @pl.when(i > 0)
    def _():
        s1_ref[...] = s1_ref[...] + ps


def _combine(q, selft, dinv, b):
    C = q.shape[0]
    dout = C * LW
    grid = (C, NB)
    in_specs = [pl.BlockSpec((1, BR, LW), lambda c, i: (c, i, 0)),
                pl.BlockSpec((1, BR, LW), lambda c, i: (c, i, 0)),
                pl.BlockSpec((BR, 1), lambda c, i: (i, 0)),
                pl.BlockSpec((1, LW), lambda c, i: (0, c))]
    out_specs = [pl.BlockSpec((BR, LW), lambda c, i: (i, c)),
                 pl.BlockSpec((1, LW), lambda c, i: (0, c))]
    out_shape = [jax.ShapeDtypeStruct((N, dout), F32),
                 jax.ShapeDtypeStruct((1, dout), F32)]
    return pl.pallas_call(
        _combine_body, grid=grid, in_specs=in_specs, out_specs=out_specs,
        out_shape=out_shape,
    )(q, selft, dinv, b.reshape(1, dout))


def _var_body(y_ref, m_ref, s2_ref):
    i = pl.program_id(1)
    d = y_ref[...] - m_ref[...]
    ps = jnp.sum(jnp.abs(d) ** 2, axis=0, keepdims=True)

    @pl.when(i == 0)
    def _():
        s2_ref[...] = ps

    @pl.when(i > 0)
    def _():
        s2_ref[...] = s2_ref[...] + ps


def _var_sums(Y, m):
    dout = Y.shape[1]
    C = dout // LW
    grid = (C, NB)
    return pl.pallas_call(
        _var_body, grid=grid,
        in_specs=[pl.BlockSpec((BR, LW), lambda c, i: (i, c)),
                  pl.BlockSpec((1, LW), lambda c, i: (0, c))],
        out_specs=pl.BlockSpec((1, LW), lambda c, i: (0, c)),
        out_shape=jax.ShapeDtypeStruct((1, dout), F32),
    )(Y, m)


def _gate_body(y_ref, m_ref, v_ref, g_ref, be_ref, gw_ref, gb_ref, b_ref,
               h_ref, gate_ref, mg_ref):
    i = pl.program_id(0)
    t = g_ref[...] * (y_ref[...] - m_ref[...])
    hb = t * lax.rsqrt(v_ref[...] + EPS) + be_ref[...]
    h_ref[...] = hb
    gate = jnp.dot(hb, gw_ref[...], preferred_element_type=F32) + gb_ref[...]
    gate_ref[...] = gate
    iota = lax.broadcasted_iota(jnp.int32, (BR, G), 1)
    mask = b_ref[...] == iota
    gv = jnp.where(mask, gate, -jnp.inf)
    bm = jnp.max(gv, axis=0, keepdims=True)

    @pl.when(i == 0)
    def _():
        mg_ref[...] = bm

    @pl.when(i > 0)
    def _():
        mg_ref[...] = jnp.maximum(mg_ref[...], bm)


def _gate(Y5, m, v, g, be, gW, gb, batch2):
    grid = (NB,)
    in_specs = [pl.BlockSpec((BR, 1024), lambda i: (i, 0)),
                pl.BlockSpec((1, 1024), lambda i: (0, 0)),
                pl.BlockSpec((1, 1024), lambda i: (0, 0)),
                pl.BlockSpec((1, 1024), lambda i: (0, 0)),
                pl.BlockSpec((1, 1024), lambda i: (0, 0)),
                pl.BlockSpec((1024, 1), lambda i: (0, 0)),
                pl.BlockSpec((1, 1), lambda i: (0, 0)),
                pl.BlockSpec((BR, 1), lambda i: (i, 0))]
    out_specs = [pl.BlockSpec((BR, 1024), lambda i: (i, 0)),
                 pl.BlockSpec((BR, 1), lambda i: (i, 0)),
                 pl.BlockSpec((1, G), lambda i: (0, 0))]
    out_shape = [jax.ShapeDtypeStruct((N, 1024), F32),
                 jax.ShapeDtypeStruct((N, 1), F32),
                 jax.ShapeDtypeStruct((1, G), F32)]
    return pl.pallas_call(
        _gate_body, grid=grid, in_specs=in_specs, out_specs=out_specs,
        out_shape=out_shape,
    )(Y5, m, v, g, be, gW, gb.reshape(1, 1), batch2)


def _exp_body(gate_ref, b_ref, mg_ref, e_ref):
    iota = lax.broadcasted_iota(jnp.int32, (BR, G), 1)
    mask = b_ref[...] == iota
    msel = jnp.max(jnp.where(mask, mg_ref[...], -jnp.inf), axis=1, keepdims=True)
    e_ref[...] = jnp.exp(gate_ref[...] - msel)


def _exp(gate, batch2, mg):
    return pl.pallas_call(
        _exp_body, grid=(NB,),
        in_specs=[pl.BlockSpec((BR, 1), lambda i: (i, 0)),
                  pl.BlockSpec((BR, 1), lambda i: (i, 0)),
                  pl.BlockSpec((1, G), lambda i: (0, 0))],
        out_specs=pl.BlockSpec((BR, 1), lambda i: (i, 0)),
        out_shape=jax.ShapeDtypeStruct((N, 1), F32),
    )(gate, batch2, mg)


def _wvals_body(e_ref, b_ref, d_ref, h_ref, o_ref):
    iota = lax.broadcasted_iota(jnp.int32, (BR, G), 1)
    mask = b_ref[...] == iota
    dsel = jnp.max(jnp.where(mask, d_ref[...], -jnp.inf), axis=1, keepdims=True)
    a = e_ref[...] / dsel
    o_ref[0] = a * h_ref[...]


def _wvals(e, batch2, d, h5):
    C = 1024 // LW
    grid = (NB, C)
    return pl.pallas_call(
        _wvals_body, grid=grid,
        in_specs=[pl.BlockSpec((BR, 1), lambda i, c: (i, 0)),
                  pl.BlockSpec((BR, 1), lambda i, c: (i, 0)),
                  pl.BlockSpec((1, G), lambda i, c: (0, 0)),
                  pl.BlockSpec((BR, LW), lambda i, c: (i, c))],
        out_specs=pl.BlockSpec((1, BR, LW), lambda i, c: (c, i, 0)),
        out_shape=jax.ShapeDtypeStruct((C, N, LW), F32),
    )(e, batch2, d, h5)


def _mlp_body(p_ref, w6_ref, b6_ref, w7_ref, b7_ref, w8_ref, b8_ref, o_ref):
    r = jnp.maximum(jnp.dot(p_ref[...], w6_ref[...],
                            preferred_element_type=F32) + b6_ref[...], 0.0)
    r = jnp.maximum(jnp.dot(r, w7_ref[...],
                            preferred_element_type=F32) + b7_ref[...], 0.0)
    o_ref[...] = jnp.dot(r, w8_ref[...], preferred_element_type=F32) + b8_ref[...]


def _mlp(pooled, W6, b6, W7, b7, W8, b8):
    full = lambda shape: pl.BlockSpec(shape, lambda: tuple(0 for _ in shape))
    return pl.pallas_call(
        _mlp_body, grid=(),
        in_specs=[full((G, 1024)), full((1024, 128)), full((1, 128)),
                  full((128, 16)), full((1, 16)), full((16, 1)), full((1, 1))],
        out_specs=full((G, 1)),
        out_shape=jax.ShapeDtypeStruct((G, 1), F32),
    )(pooled, W6, b6.reshape(1, 128), W7, b7.reshape(1, 16), W8,
      b8.reshape(1, 1))


NA_PAD = 10240
NA_ROWS = 144
NCT_MAIN, RPC_MAIN = 10, 1000
NCT_ATT, RPC_ATT = 9, 16


def kernel(x, edge_index, batch, W1, b1, g1, be1, W2, b2, g2, be2, W3, b3, g3,
           be3, W4, b4, g4, be4, W5, b5, g5, be5, gW, gb, W6, b6, W7, b7, W8, b8):
    src, dst = edge_index[0], edge_index[1]
    order = jnp.argsort(dst, stable=True)
    dstp = dst[order]
    srcp = src[order]
    seg_start = jnp.searchsorted(dstp, jnp.arange(N + 1, dtype=jnp.int32))
    deg = (seg_start[1:] - seg_start[:-1] + 1).astype(F32)
    dinv = 1.0 / jnp.sqrt(deg)
    dinv2 = dinv[:, None]
    nself = (dinv * dinv)[:, None]

    srcp3 = srcp.reshape(NS, E // NS // K_EDGE, K_EDGE)
    dstp3 = dstp.reshape(NS, E // NS // K_EDGE, K_EDGE)
    zeros_main = jnp.zeros((RPC_MAIN, LW), F32)
    zeros_att = jnp.zeros((RPC_ATT, LW), F32)

    params = [(W1, b1, g1, be1), (W2, b2, g2, be2), (W3, b3, g3, be3),
              (W4, b4, g4, be4), (W5, b5, g5, be5)]
    Y = x
    m = v = gg = bb = None
    for li, (W, b, g, be) in enumerate(params):
        first = li == 0
        dzero = jnp.zeros((1, W.shape[0]), F32)
        p, selft = _mm_scale(Y,
                             dzero if first else m,
                             dzero if first else v,
                             dzero if first else gg.reshape(1, -1),
                             dzero if first else bb.reshape(1, -1),
                             W, dinv2, nself, first)
        C = W.shape[1] // LW
        agg = _make_sc_agg(C, N, NCT_MAIN, RPC_MAIN, E)
        q = agg(p, srcp3, dstp3, zeros_main)
        Y, S1 = _combine(q, selft, dinv2, b)
        m = S1 / float(N)
        S2 = _var_sums(Y, m)
        v = S2 / float(N)
        gg, bb = g, be

    batch2 = batch[:, None]
    h5, gate, mg = _gate(Y, m, v, gg.reshape(1, -1), bb.reshape(1, -1),
                         gW, gb, batch2)
    e = _exp(gate, batch2, mg)

    batch_pad = jnp.concatenate(
        [batch, jnp.full((NA_PAD - N,), NA_ROWS - 1, batch.dtype)])
    batch_pad3 = batch_pad.reshape(NS, NA_PAD // NS // K_EDGE, K_EDGE)
    iota_pad3 = jnp.arange(NA_PAD, dtype=jnp.int32).reshape(
        NS, NA_PAD // NS // K_EDGE, K_EDGE)

    e_wide = jnp.pad(jnp.broadcast_to(e, (N, LW)), ((0, NA_PAD - N), (0, 0)))
    agg_d = _make_sc_agg(1, NA_ROWS, NCT_ATT, RPC_ATT, NA_PAD)
    dsum = agg_d(e_wide[None], iota_pad3, batch_pad3, zeros_att)
    d_row = dsum[0, :G, :1].reshape(1, G)

    wv = _wvals(e, batch2, d_row, h5)
    wv_pad = jnp.pad(wv, ((0, 0), (0, NA_PAD - N), (0, 0)))
    agg_s = _make_sc_agg(1024 // LW, NA_ROWS, NCT_ATT, RPC_ATT, NA_PAD)
    s = agg_s(wv_pad, iota_pad3, batch_pad3, zeros_att)
    pooled = jnp.transpose(s[:, :G, :], (1, 0, 2)).reshape(G, 1024)

    return _mlp(pooled, W6, b6, W7, b7, W8, b8)


# speedup vs baseline: 4.0660x; 3.1454x over previous
import functools
import jax
import jax.numpy as jnp
from jax import lax
from jax.experimental import pallas as pl
from jax.experimental.pallas import tpu as pltpu
from jax.experimental.pallas import tpu_sc as plsc

N = 10000
E = 160000
G = 128
EPS = 1e-5
F32 = jnp.float32
LW = 128
BR = 2000
NB = N // BR
K_EDGE = 80
NS = 16


def _sc_agg_body(C, n_rows, n_copy_tiles, rpc, e_pad, per_tile,
                 p_hbm, srcp3, dstp3, zeros_hbm, out,
                 src_t, dst0, dst1, rows0, rows1, sem0, sem1, semd0, semd1, acc):
    core = lax.axis_index("c")
    tile = lax.axis_index("s")
    row0 = tile * rpc
    nj = per_tile // K_EDGE
    rows = (rows0, rows1)
    sems = (sem0, sem1)
    dsts = (dst0, dst1)
    semd = (semd0, semd1)

    pltpu.sync_copy(srcp3.at[tile], src_t)

    def process(cc):
        @pl.when(tile < n_copy_tiles)
        def _():
            pltpu.sync_copy(zeros_hbm, acc.at[pl.ds(row0, rpc)])
        plsc.subcore_barrier()

        def issue(j, b):
            pltpu.async_copy(p_hbm.at[cc].at[src_t.at[j]], rows[b], sems[b])
            pltpu.async_copy(dstp3.at[tile].at[j], dsts[b], semd[b])

        def wait(b):
            pltpu.make_async_copy(p_hbm.at[cc].at[src_t.at[0]],
                                  rows[b], sems[b]).wait()
            pltpu.make_async_copy(dstp3.at[tile].at[0], dsts[b], semd[b]).wait()

        def step(j, b):
            @pl.when(j + 1 < nj)
            def _():
                issue(j + 1, 1 - b)
            wait(b)
            pltpu.sync_copy(rows[b], acc.at[dsts[b]], add=True)

        issue(0, 0)

        def dbl(i, carry):
            step(i * 2, 0)
            step(i * 2 + 1, 1)
            return carry

        lax.fori_loop(0, nj // 2, dbl, 0)
        if nj % 2:
            step(nj - 1, 0)
        plsc.subcore_barrier()

        @pl.when(tile < n_copy_tiles)
        def _():
            pltpu.sync_copy(acc.at[pl.ds(row0, rpc)],
                            out.at[cc].at[pl.ds(row0, rpc)])
        plsc.subcore_barrier()

    for idx in range((C + 1) // 2):
        for co in (0, 1):
            cc = idx * 2 + co
            if cc >= C:
                continue

            @pl.when(core == co)
            def _(cc=cc):
                process(cc)


def _make_sc_agg(C, n_rows, n_copy_tiles, rpc, e_pad):
    per_tile = e_pad // NS
    nj = per_tile // K_EDGE
    mesh = plsc.VectorSubcoreMesh(core_axis_name="c", subcore_axis_name="s")
    body = functools.partial(_sc_agg_body, C, n_rows, n_copy_tiles, rpc, e_pad, per_tile)
    return pl.kernel(
        body,
        out_type=jax.ShapeDtypeStruct((C, n_rows, LW), F32),
        mesh=mesh,
        scratch_types=[
            pltpu.VMEM((nj, K_EDGE), jnp.int32),
            pltpu.VMEM((K_EDGE,), jnp.int32),
            pltpu.VMEM((K_EDGE,), jnp.int32),
            pltpu.VMEM((K_EDGE, LW), F32),
            pltpu.VMEM((K_EDGE, LW), F32),
            pltpu.SemaphoreType.DMA,
            pltpu.SemaphoreType.DMA,
            pltpu.SemaphoreType.DMA,
            pltpu.SemaphoreType.DMA,
            pltpu.VMEM_SHARED((n_rows, LW), F32),
        ],
    )


def _mm_scale_body(first, din, x_ref, m_ref, v_ref, g_ref, be_ref, w_ref,
                   dinv_ref, nself_ref, p_ref, self_ref):
    xb = x_ref[...]
    if not first:
        t = g_ref[...] * (xb - m_ref[...])
        xb = t * lax.rsqrt(v_ref[...] + EPS) + be_ref[...]
    hw = jnp.dot(xb, w_ref[...], preferred_element_type=F32)
    p_ref[0] = dinv_ref[...] * hw
    self_ref[0] = hw * nself_ref[...]


def _mm_scale(Y, m, v, g, be, W, dinv, nself, first):
    din, dout = W.shape
    C = dout // LW
    grid = (NB, C)
    in_specs = [pl.BlockSpec((BR, din), lambda i, c: (i, 0)),
                pl.BlockSpec((1, din), lambda i, c: (0, 0)),
                pl.BlockSpec((1, din), lambda i, c: (0, 0)),
                pl.BlockSpec((1, din), lambda i, c: (0, 0)),
                pl.BlockSpec((1, din), lambda i, c: (0, 0)),
                pl.BlockSpec((din, LW), lambda i, c: (0, c)),
                pl.BlockSpec((BR, 1), lambda i, c: (i, 0)),
                pl.BlockSpec((BR, 1), lambda i, c: (i, 0))]
    out_specs = [pl.BlockSpec((1, BR, LW), lambda i, c: (c, i, 0)),
                 pl.BlockSpec((1, BR, LW), lambda i, c: (c, i, 0))]
    out_shape = [jax.ShapeDtypeStruct((C, N, LW), F32),
                 jax.ShapeDtypeStruct((C, N, LW), F32)]
    return pl.pallas_call(
        functools.partial(_mm_scale_body, first, din),
        grid=grid, in_specs=in_specs, out_specs=out_specs, out_shape=out_shape,
    )(Y, m, v, g, be, W, dinv, nself)


def _combine_body(q_ref, s_ref, dinv_ref, b_ref, y_ref, s1_ref):
    i = pl.program_id(1)
    t = dinv_ref[...] * q_ref[0]
    t = t + s_ref[0]
    y = jnp.maximum(t + b_ref[...], 0.0)
    y_ref[...] = y
    ps = jnp.sum(y, axis=0, keepdims=True)

    @pl.when(i == 0)
    def _():
        s1_ref[...] = ps

    @pl.when(i > 0)
    def _():
        s1_ref[...] = s1_ref[...] + ps


def _combine(q, selft, dinv, b):
    C = q.shape[0]
    dout = C * LW
    grid = (C, NB)
    in_specs = [pl.BlockSpec((1, BR, LW), lambda c, i: (c, i, 0)),
                pl.BlockSpec((1, BR, LW), lambda c, i: (c, i, 0)),
                pl.BlockSpec((BR, 1), lambda c, i: (i, 0)),
                pl.BlockSpec((1, LW), lambda c, i: (0, c))]
    out_specs = [pl.BlockSpec((BR, LW), lambda c, i: (i, c)),
                 pl.BlockSpec((1, LW), lambda c, i: (0, c))]
    out_shape = [jax.ShapeDtypeStruct((N, dout), F32),
                 jax.ShapeDtypeStruct((1, dout), F32)]
    return pl.pallas_call(
        _combine_body, grid=grid, in_specs=in_specs, out_specs=out_specs,
        out_shape=out_shape,
    )(q, selft, dinv, b.reshape(1, dout))


def _var_body(y_ref, m_ref, s2_ref):
    i = pl.program_id(1)
    d = y_ref[...] - m_ref[...]
    ps = jnp.sum(jnp.abs(d) ** 2, axis=0, keepdims=True)

    @pl.when(i == 0)
    def _():
        s2_ref[...] = ps

    @pl.when(i > 0)
    def _():
        s2_ref[...] = s2_ref[...] + ps


def _var_sums(Y, m):
    dout = Y.shape[1]
    C = dout // LW
    grid = (C, NB)
    return pl.pallas_call(
        _var_body, grid=grid,
        in_specs=[pl.BlockSpec((BR, LW), lambda c, i: (i, c)),
                  pl.BlockSpec((1, LW), lambda c, i: (0, c))],
        out_specs=pl.BlockSpec((1, LW), lambda c, i: (0, c)),
        out_shape=jax.ShapeDtypeStruct((1, dout), F32),
    )(Y, m)


def _gate_body(y_ref, m_ref, v_ref, g_ref, be_ref, gw_ref, gb_ref, b_ref,
               h_ref, gate_ref, mg_ref):
    i = pl.program_id(0)
    t = g_ref[...] * (y_ref[...] - m_ref[...])
    hb = t * lax.rsqrt(v_ref[...] + EPS) + be_ref[...]
    h_ref[...] = hb
    gate = jnp.dot(hb, gw_ref[...], preferred_element_type=F32) + gb_ref[...]
    gate_ref[...] = gate
    iota = lax.broadcasted_iota(jnp.int32, (BR, G), 1)
    mask = b_ref[...] == iota
    gv = jnp.where(mask, gate, -jnp.inf)
    bm = jnp.max(gv, axis=0, keepdims=True)

    @pl.when(i == 0)
    def _():
        mg_ref[...] = bm

    @pl.when(i > 0)
    def _():
        mg_ref[...] = jnp.maximum(mg_ref[...], bm)


def _gate(Y5, m, v, g, be, gW, gb, batch2):
    grid = (NB,)
    in_specs = [pl.BlockSpec((BR, 1024), lambda i: (i, 0)),
                pl.BlockSpec((1, 1024), lambda i: (0, 0)),
                pl.BlockSpec((1, 1024), lambda i: (0, 0)),
                pl.BlockSpec((1, 1024), lambda i: (0, 0)),
                pl.BlockSpec((1, 1024), lambda i: (0, 0)),
                pl.BlockSpec((1024, 1), lambda i: (0, 0)),
                pl.BlockSpec((1, 1), lambda i: (0, 0)),
                pl.BlockSpec((BR, 1), lambda i: (i, 0))]
    out_specs = [pl.BlockSpec((BR, 1024), lambda i: (i, 0)),
                 pl.BlockSpec((BR, 1), lambda i: (i, 0)),
                 pl.BlockSpec((1, G), lambda i: (0, 0))]
    out_shape = [jax.ShapeDtypeStruct((N, 1024), F32),
                 jax.ShapeDtypeStruct((N, 1), F32),
                 jax.ShapeDtypeStruct((1, G), F32)]
    return pl.pallas_call(
        _gate_body, grid=grid, in_specs=in_specs, out_specs=out_specs,
        out_shape=out_shape,
    )(Y5, m, v, g, be, gW, gb.reshape(1, 1), batch2)


def _exp_body(gate_ref, b_ref, mg_ref, e_ref):
    iota = lax.broadcasted_iota(jnp.int32, (BR, G), 1)
    mask = b_ref[...] == iota
    msel = jnp.max(jnp.where(mask, mg_ref[...], -jnp.inf), axis=1, keepdims=True)
    e_ref[...] = jnp.exp(gate_ref[...] - msel)


def _exp(gate, batch2, mg):
    return pl.pallas_call(
        _exp_body, grid=(NB,),
        in_specs=[pl.BlockSpec((BR, 1), lambda i: (i, 0)),
                  pl.BlockSpec((BR, 1), lambda i: (i, 0)),
                  pl.BlockSpec((1, G), lambda i: (0, 0))],
        out_specs=pl.BlockSpec((BR, 1), lambda i: (i, 0)),
        out_shape=jax.ShapeDtypeStruct((N, 1), F32),
    )(gate, batch2, mg)


def _wvals_body(e_ref, b_ref, d_ref, h_ref, o_ref):
    iota = lax.broadcasted_iota(jnp.int32, (BR, G), 1)
    mask = b_ref[...] == iota
    dsel = jnp.max(jnp.where(mask, d_ref[...], -jnp.inf), axis=1, keepdims=True)
    a = e_ref[...] / dsel
    o_ref[0] = a * h_ref[...]


def _wvals(e, batch2, d, h5):
    C = 1024 // LW
    grid = (NB, C)
    return pl.pallas_call(
        _wvals_body, grid=grid,
        in_specs=[pl.BlockSpec((BR, 1), lambda i, c: (i, 0)),
                  pl.BlockSpec((BR, 1), lambda i, c: (i, 0)),
                  pl.BlockSpec((1, G), lambda i, c: (0, 0)),
                  pl.BlockSpec((BR, LW), lambda i, c: (i, c))],
        out_specs=pl.BlockSpec((1, BR, LW), lambda i, c: (c, i, 0)),
        out_shape=jax.ShapeDtypeStruct((C, N, LW), F32),
    )(e, batch2, d, h5)


def _mlp_body(p_ref, w6_ref, b6_ref, w7_ref, b7_ref, w8_ref, b8_ref, o_ref):
    r = jnp.maximum(jnp.dot(p_ref[...], w6_ref[...],
                            preferred_element_type=F32) + b6_ref[...], 0.0)
    r = jnp.maximum(jnp.dot(r, w7_ref[...],
                            preferred_element_type=F32) + b7_ref[...], 0.0)
    o_ref[...] = jnp.dot(r, w8_ref[...], preferred_element_type=F32) + b8_ref[...]


def _mlp(pooled, W6, b6, W7, b7, W8, b8):
    full = lambda shape: pl.BlockSpec(shape, lambda: tuple(0 for _ in shape))
    return pl.pallas_call(
        _mlp_body, grid=(),
        in_specs=[full((G, 1024)), full((1024, 128)), full((1, 128)),
                  full((128, 16)), full((1, 16)), full((16, 1)), full((1, 1))],
        out_specs=full((G, 1)),
        out_shape=jax.ShapeDtypeStruct((G, 1), F32),
    )(pooled, W6, b6.reshape(1, 128), W7, b7.reshape(1, 16), W8,
      b8.reshape(1, 1))


NA_PAD = 10240
NA_ROWS = 144
NCT_MAIN, RPC_MAIN = 10, 1000
NCT_ATT, RPC_ATT = 9, 16


def kernel(x, edge_index, batch, W1, b1, g1, be1, W2, b2, g2, be2, W3, b3, g3,
           be3, W4, b4, g4, be4, W5, b5, g5, be5, gW, gb, W6, b6, W7, b7, W8, b8):
    src, dst = edge_index[0], edge_index[1]
    order = jnp.argsort(dst, stable=True)
    dstp = dst[order]
    srcp = src[order]
    srcp3_pre = srcp.reshape(NS, E // NS // K_EDGE, K_EDGE)
    dstp3_pre = dstp.reshape(NS, E // NS // K_EDGE, K_EDGE)
    zeros_pre = jnp.zeros((RPC_MAIN, LW), F32)
    ones_p = jnp.ones((1, N, LW), F32)
    agg_deg = _make_sc_agg(1, N, NCT_MAIN, RPC_MAIN, E)
    degq = agg_deg(ones_p, srcp3_pre, dstp3_pre, zeros_pre)
    deg = degq[0, :, 0] + 1.0
    dinv = 1.0 / jnp.sqrt(deg)
    dinv2 = dinv[:, None]
    nself = (dinv * dinv)[:, None]

    srcp3 = srcp.reshape(NS, E // NS // K_EDGE, K_EDGE)
    dstp3 = dstp.reshape(NS, E // NS // K_EDGE, K_EDGE)
    zeros_main = jnp.zeros((RPC_MAIN, LW), F32)
    zeros_att = jnp.zeros((RPC_ATT, LW), F32)

    params = [(W1, b1, g1, be1), (W2, b2, g2, be2), (W3, b3, g3, be3),
              (W4, b4, g4, be4), (W5, b5, g5, be5)]
    Y = x
    m = v = gg = bb = None
    for li, (W, b, g, be) in enumerate(params):
        first = li == 0
        dzero = jnp.zeros((1, W.shape[0]), F32)
        p, selft = _mm_scale(Y,
                             dzero if first else m,
                             dzero if first else v,
                             dzero if first else gg.reshape(1, -1),
                             dzero if first else bb.reshape(1, -1),
                             W, dinv2, nself, first)
        C = W.shape[1] // LW
        agg = _make_sc_agg(C, N, NCT_MAIN, RPC_MAIN, E)
        q = agg(p, srcp3, dstp3, zeros_main)
        Y, S1 = _combine(q, selft, dinv2, b)
        m = S1 / float(N)
        S2 = _var_sums(Y, m)
        v = S2 / float(N)
        gg, bb = g, be

    batch2 = batch[:, None]
    h5, gate, mg = _gate(Y, m, v, gg.reshape(1, -1), bb.reshape(1, -1),
                         gW, gb, batch2)
    e = _exp(gate, batch2, mg)

    batch_pad = jnp.concatenate(
        [batch, jnp.full((NA_PAD - N,), NA_ROWS - 1, batch.dtype)])
    batch_pad3 = batch_pad.reshape(NS, NA_PAD // NS // K_EDGE, K_EDGE)
    iota_pad3 = jnp.arange(NA_PAD, dtype=jnp.int32).reshape(
        NS, NA_PAD // NS // K_EDGE, K_EDGE)

    e_wide = jnp.pad(jnp.broadcast_to(e, (N, LW)), ((0, NA_PAD - N), (0, 0)))
    agg_d = _make_sc_agg(1, NA_ROWS, NCT_ATT, RPC_ATT, NA_PAD)
    dsum = agg_d(e_wide[None], iota_pad3, batch_pad3, zeros_att)
    d_row = dsum[0, :G, :1].reshape(1, G)

    wv = _wvals(e, batch2, d_row, h5)
    wv_pad = jnp.pad(wv, ((0, 0), (0, NA_PAD - N), (0, 0)))
    agg_s = _make_sc_agg(1024 // LW, NA_ROWS, NCT_ATT, RPC_ATT, NA_PAD)
    s = agg_s(wv_pad, iota_pad3, batch_pad3, zeros_att)
    pooled = jnp.transpose(s[:, :G, :], (1, 0, 2)).reshape(G, 1024)

    return _mlp(pooled, W6, b6, W7, b7, W8, b8)
